# baseline (speedup 1.0000x reference)
import functools

import jax
import jax.numpy as jnp
from jax import lax
from jax.experimental import pallas as pl
from jax.experimental.pallas import tpu as pltpu
from jax.experimental.pallas import tpu_sc as plsc


def _make_sc_gather(n_entries_table, n_rows_total, n_cond, chunk,
                    n_chunks_per_worker, n_workers, n_cores):
    mesh = plsc.VectorSubcoreMesh(core_axis_name="c", subcore_axis_name="s")
    rows_per_worker = chunk * n_chunks_per_worker
    epw = 2 * rows_per_worker
    epc = 2 * chunk
    phase_step = chunk % n_cond

    @functools.partial(
        pl.kernel,
        out_type=jax.ShapeDtypeStruct((2 * n_rows_total, 16), jnp.float32),
        mesh=mesh,
        scratch_types=[
            pltpu.VMEM((epw // 128, 128), jnp.int32),
            pltpu.VMEM(((n_cond + chunk) * 32,), jnp.float32),
            pltpu.VMEM((epc, 16), jnp.float32),
            pltpu.SemaphoreType.DMA,
        ],
        compiler_params=pltpu.CompilerParams(use_tc_tiling_on_sc=False),
    )
    def sc_kernel(ids_hbm, table_hbm, pat_hbm, out_hbm, idx_v, pat_v, gbuf,
                  sem):
        wid = lax.axis_index("s") * n_cores + lax.axis_index("c")
        pltpu.sync_copy(ids_hbm.at[wid], idx_v)
        pltpu.sync_copy(pat_hbm, pat_v)

        def chunk_body(g, carry):
            d1 = pltpu.async_copy(table_hbm.at[idx_v.at[2 * g]],
                                  gbuf.at[pl.ds(0, 128)], sem)
            d2 = pltpu.async_copy(table_hbm.at[idx_v.at[2 * g + 1]],
                                  gbuf.at[pl.ds(128, 128)], sem)
            d1.wait()
            d2.wait()

            p = lax.rem(g * phase_step, n_cond) * 32

            def add_body(rr, c2):
                plsc.addupdate(gbuf.at[rr, pl.ds(0, 16)],
                               pat_v[pl.ds(p + rr * 16, 16)])
                return c2

            lax.fori_loop(0, epc, add_body, 0, unroll=8)

            pltpu.sync_copy(gbuf, out_hbm.at[pl.ds(wid * epw + g * epc, epc)])
            return carry

        lax.fori_loop(0, n_chunks_per_worker, chunk_body, 0)

    return sc_kernel


def kernel(cat_ids, cond_table, cat_table):
    b, n_cond = cat_ids.shape
    dim = cat_table.shape[1]
    n_cat = cat_table.shape[0] // n_cond

    info = plsc.get_sparse_core_info()
    n_cores, n_subcores = info.num_cores, info.num_subcores
    n_workers = n_cores * n_subcores

    n_rows = b * n_cond
    chunk = 128
    rows_per_worker = n_rows // n_workers
    n_chunks_per_worker = rows_per_worker // chunk
    assert rows_per_worker % chunk == 0

    offsets = jnp.arange(n_cond, dtype=jnp.int32) * n_cat
    flat_ids = (cat_ids.astype(jnp.int32) + offsets[None, :]).reshape(-1)
    ids_dbl = (flat_ids[:, None] * 2 +
               jnp.arange(2, dtype=jnp.int32)[None, :]).reshape(
                   n_workers, 2 * rows_per_worker // 128, 128)
    reps = (n_cond + chunk + n_cond - 1) // n_cond
    pat = jnp.tile(cond_table[1:n_cond + 1],
                   (reps, 1)).reshape(-1)[:(n_cond + chunk) * dim]

    sc_gather = _make_sc_gather(2 * cat_table.shape[0], n_rows, n_cond, chunk,
                                n_chunks_per_worker, n_workers, n_cores)
    out = sc_gather(ids_dbl, cat_table.reshape(-1, 16), pat)
    return out.reshape(b, n_cond, dim)
